# Initial kernel scaffold; baseline (speedup 1.0000x reference)
import functools

import jax
import jax.numpy as jnp
from jax import lax
from jax.experimental import pallas as pl
from jax.experimental.pallas import tpu as pltpu
from jax.experimental.pallas import tpu_sc as plsc

N = 10000
F = 128
H = 16
E = 320000

NC, NS = 2, 16
NW = NC * NS
NP = 10240
SLAB = NP // NS

CH = 128
GRP = 8
CPT = 80
EPT = CPT * CH
EPAD = NW * EPT
ROWS2D = EPAD // CH


def _sc_scatter_body(with_gather, table_hbm, src2d, dst2d, zeros_hbm,
                     ones_hbm, out_hbm, ridx_v, cidx_v, rows_v, gsem):
  cid = lax.axis_index("c")
  sid = lax.axis_index("s")
  wid = sid * NC + cid

  def body(acc):
    pltpu.sync_copy(zeros_hbm.at[pl.ds(sid * SLAB, SLAB)],
                    acc.at[pl.ds(sid * SLAB, SLAB)])
    if with_gather:
      pltpu.sync_copy(src2d.at[pl.ds(wid * CPT, CPT)], ridx_v)
    else:
      for b in range(GRP):
        pltpu.sync_copy(ones_hbm, rows_v.at[pl.ds(b * CH, CH)])
    pltpu.sync_copy(dst2d.at[pl.ds(wid * CPT, CPT)], cidx_v)
    plsc.subcore_barrier()

    def group(g, _):
      if with_gather:
        cps = []
        for b in range(GRP):
          cps.append(pltpu.async_copy(
              table_hbm.at[ridx_v.at[g * GRP + b]],
              rows_v.at[pl.ds(b * CH, CH)], gsem.at[b]))
        for b in range(GRP):
          cps[b].wait()
          pltpu.sync_copy(rows_v.at[pl.ds(b * CH, CH)],
                          acc.at[cidx_v.at[g * GRP + b]], add=True)
      else:
        for b in range(GRP):
          pltpu.sync_copy(rows_v.at[pl.ds(b * CH, CH)],
                          acc.at[cidx_v.at[g * GRP + b]], add=True)
      return 0

    lax.fori_loop(0, CPT // GRP, group, 0)

    plsc.subcore_barrier()
    pltpu.sync_copy(acc.at[pl.ds(sid * SLAB, SLAB)],
                    out_hbm.at[cid, pl.ds(sid * SLAB, SLAB)])

  pl.run_scoped(body, pltpu.VMEM_SHARED((NP, H), jnp.float32))


def _make_sc_pass(with_gather):
  mesh = plsc.VectorSubcoreMesh(core_axis_name="c", subcore_axis_name="s")
  return pl.kernel(
      functools.partial(_sc_scatter_body, with_gather),
      out_type=jax.ShapeDtypeStruct((NC, NP, H), jnp.float32),
      mesh=mesh,
      scratch_types=[
          pltpu.VMEM((CPT, CH), jnp.int32),
          pltpu.VMEM((CPT, CH), jnp.int32),
          pltpu.VMEM((GRP * CH, H), jnp.float32),
          pltpu.SemaphoreType.DMA((GRP,)),
      ],
  )


_sc_gather_scatter = _make_sc_pass(True)
_sc_degree = _make_sc_pass(False)


def _tc_a(x_ref, w0_ref, w1_ref, b1_ref, xw10_ref, xw11_ref):
  x = x_ref[...]
  xw10_ref[...] = jnp.dot(x, w0_ref[...],
                          preferred_element_type=jnp.float32) + b1_ref[...]
  xw11_ref[...] = jnp.dot(x, w1_ref[...], preferred_element_type=jnp.float32)


def _tc_b(degp_ref, xw11_ref, dis_ref, a1_ref):
  deg = degp_ref[0] + degp_ref[1]
  dis = jnp.where(deg > 0.0, lax.rsqrt(deg), 0.0)
  dis_ref[...] = dis
  a1_ref[...] = dis * xw11_ref[...]


def _tc_c(xw10_ref, dis_ref, g1p_ref, h_ref, a2_ref):
  g1 = g1p_ref[0] + g1p_ref[1]
  h = jnp.maximum(xw10_ref[...] - dis_ref[...] * g1, 0.0)
  h_ref[...] = h
  a2_ref[...] = dis_ref[...] * h


def _tc_d(h_ref, dis_ref, g2p_ref, w20_ref, w21_ref, b2_ref, out_ref):
  t = -(dis_ref[...] * (g2p_ref[0] + g2p_ref[1]))
  o = (jnp.dot(h_ref[...], w20_ref[...], preferred_element_type=jnp.float32)
       + jnp.dot(t, w21_ref[...], preferred_element_type=jnp.float32)
       + b2_ref[...])
  m = jnp.max(o, axis=1, keepdims=True)
  s = jnp.sum(jnp.exp(o - m), axis=1, keepdims=True)
  out_ref[...] = (o - m) - jnp.log(s)


def kernel(x, edge_index, W1_0, W1_1, b1, W2_0, W2_1, b2):
  x_pad = jnp.concatenate(
      [x, jnp.zeros((NP - N, F), jnp.float32)], axis=0)
  epad = jnp.full((2, EPAD - E), N, jnp.int32)
  ei = jnp.concatenate([edge_index, epad], axis=1)
  row2d = ei[0].reshape(ROWS2D, CH)
  col2d = ei[1].reshape(ROWS2D, CH)
  zeros_np = jnp.zeros((NP, H), jnp.float32)
  ones_ch = jnp.ones((CH, H), jnp.float32)
  b1r = b1.reshape(1, H)
  b2r = b2.reshape(1, F)

  xw10, xw11 = pl.pallas_call(
      _tc_a,
      out_shape=(jax.ShapeDtypeStruct((NP, H), jnp.float32),
                 jax.ShapeDtypeStruct((NP, H), jnp.float32)),
  )(x_pad, W1_0, W1_1, b1r)

  degp = _sc_degree(zeros_np, row2d, row2d, zeros_np, ones_ch)

  dis, a1 = pl.pallas_call(
      _tc_b,
      out_shape=(jax.ShapeDtypeStruct((NP, H), jnp.float32),
                 jax.ShapeDtypeStruct((NP, H), jnp.float32)),
  )(degp, xw11)

  g1p = _sc_gather_scatter(a1, row2d, col2d, zeros_np, ones_ch)

  h, a2 = pl.pallas_call(
      _tc_c,
      out_shape=(jax.ShapeDtypeStruct((NP, H), jnp.float32),
                 jax.ShapeDtypeStruct((NP, H), jnp.float32)),
  )(xw10, dis, g1p)

  g2p = _sc_gather_scatter(a2, row2d, col2d, zeros_np, ones_ch)

  out = pl.pallas_call(
      _tc_d,
      out_shape=jax.ShapeDtypeStruct((NP, F), jnp.float32),
  )(h, dis, g2p, W2_0, W2_1, b2r)

  return out[:N]


# speedup vs baseline: 28.7208x; 28.7208x over previous
import functools

import jax
import jax.numpy as jnp
from jax import lax
from jax.experimental import pallas as pl
from jax.experimental.pallas import tpu as pltpu
from jax.experimental.pallas import tpu_sc as plsc

N = 10000
F = 128
H = 16
E = 320000

NC, NS = 2, 16
NW = NC * NS
NP = 10240
SLAB = NP // NS

CH = 128
GRP = 8
CPT = 80
EPT = CPT * CH
EPAD = NW * EPT
ROWS2D = EPAD // CH


def _sc_scatter_body(with_gather, table_hbm, src2d, dst2d, zeros_hbm,
                     ones_hbm, out_hbm, ridx_v, cidx_v, rows_v, gsem, acc):
  cid = lax.axis_index("c")
  sid = lax.axis_index("s")
  wid = sid * NC + cid

  def body():
    pltpu.sync_copy(zeros_hbm.at[pl.ds(sid * SLAB, SLAB)],
                    acc.at[pl.ds(sid * SLAB, SLAB)])
    if with_gather:
      pltpu.sync_copy(src2d.at[pl.ds(wid * CPT, CPT)], ridx_v)
    else:
      for b in range(GRP):
        pltpu.sync_copy(ones_hbm, rows_v.at[pl.ds(b * CH, CH)])
    pltpu.sync_copy(dst2d.at[pl.ds(wid * CPT, CPT)], cidx_v)
    plsc.subcore_barrier()

    def group(g, _):
      if with_gather:
        cps = []
        for b in range(GRP):
          cps.append(pltpu.async_copy(
              table_hbm.at[ridx_v.at[g * GRP + b]],
              rows_v.at[pl.ds(b * CH, CH)], gsem.at[b]))
        for b in range(GRP):
          cps[b].wait()
          pltpu.sync_copy(rows_v.at[pl.ds(b * CH, CH)],
                          acc.at[cidx_v.at[g * GRP + b]], add=True)
      else:
        for b in range(GRP):
          pltpu.sync_copy(rows_v.at[pl.ds(b * CH, CH)],
                          acc.at[cidx_v.at[g * GRP + b]], add=True)
      return 0

    lax.fori_loop(0, CPT // GRP, group, 0)

    plsc.subcore_barrier()
    pltpu.sync_copy(acc.at[pl.ds(sid * SLAB, SLAB)],
                    out_hbm.at[cid, pl.ds(sid * SLAB, SLAB)])

  body()


def _make_sc_pass(with_gather):
  mesh = plsc.VectorSubcoreMesh(core_axis_name="c", subcore_axis_name="s",
                                num_cores=NC, num_subcores=NS)
  return pl.kernel(
      functools.partial(_sc_scatter_body, with_gather),
      out_type=jax.ShapeDtypeStruct((NC, NP, H), jnp.float32),
      mesh=mesh,
      scratch_types=[
          pltpu.VMEM((CPT, CH), jnp.int32),
          pltpu.VMEM((CPT, CH), jnp.int32),
          pltpu.VMEM((GRP * CH, H), jnp.float32),
          pltpu.SemaphoreType.DMA((GRP,)),
          pltpu.VMEM_SHARED((NP, H), jnp.float32),
      ],
      compiler_params=pltpu.CompilerParams(use_tc_tiling_on_sc=False),
  )


_sc_gather_scatter = _make_sc_pass(True)
_sc_degree = _make_sc_pass(False)


def _tc_a(x_ref, w0_ref, w1_ref, b1_ref, xw10_ref, xw11_ref):
  x = x_ref[...]
  xw10_ref[...] = jnp.dot(x, w0_ref[...],
                          preferred_element_type=jnp.float32) + b1_ref[...]
  xw11_ref[...] = jnp.dot(x, w1_ref[...], preferred_element_type=jnp.float32)


def _tc_b(degp_ref, xw11_ref, dis_ref, a1_ref):
  deg = degp_ref[0] + degp_ref[1]
  dis = jnp.where(deg > 0.0, lax.rsqrt(deg), 0.0)
  dis_ref[...] = dis
  a1_ref[...] = dis * xw11_ref[...]


def _tc_c(xw10_ref, dis_ref, g1p_ref, h_ref, a2_ref):
  g1 = g1p_ref[0] + g1p_ref[1]
  h = jnp.maximum(xw10_ref[...] - dis_ref[...] * g1, 0.0)
  h_ref[...] = h
  a2_ref[...] = dis_ref[...] * h


def _tc_d(h_ref, dis_ref, g2p_ref, w20_ref, w21_ref, b2_ref, out_ref):
  t = -(dis_ref[...] * (g2p_ref[0] + g2p_ref[1]))
  o = (jnp.dot(h_ref[...], w20_ref[...], preferred_element_type=jnp.float32)
       + jnp.dot(t, w21_ref[...], preferred_element_type=jnp.float32)
       + b2_ref[...])
  m = jnp.max(o, axis=1, keepdims=True)
  s = jnp.sum(jnp.exp(o - m), axis=1, keepdims=True)
  out_ref[...] = (o - m) - jnp.log(s)


def kernel(x, edge_index, W1_0, W1_1, b1, W2_0, W2_1, b2):
  x_pad = jnp.concatenate(
      [x, jnp.zeros((NP - N, F), jnp.float32)], axis=0)
  epad = jnp.full((2, EPAD - E), N, jnp.int32)
  ei = jnp.concatenate([edge_index, epad], axis=1)
  row2d = ei[0].reshape(ROWS2D, CH)
  col2d = ei[1].reshape(ROWS2D, CH)
  zeros_np = jnp.zeros((NP, H), jnp.float32)
  ones_ch = jnp.ones((CH, H), jnp.float32)
  b1r = b1.reshape(1, H)
  b2r = b2.reshape(1, F)

  xw10, xw11 = pl.pallas_call(
      _tc_a,
      out_shape=(jax.ShapeDtypeStruct((NP, H), jnp.float32),
                 jax.ShapeDtypeStruct((NP, H), jnp.float32)),
  )(x_pad, W1_0, W1_1, b1r)

  degp = _sc_degree(zeros_np, row2d, row2d, zeros_np, ones_ch)

  dis, a1 = pl.pallas_call(
      _tc_b,
      out_shape=(jax.ShapeDtypeStruct((NP, H), jnp.float32),
                 jax.ShapeDtypeStruct((NP, H), jnp.float32)),
  )(degp, xw11)

  g1p = _sc_gather_scatter(a1, row2d, col2d, zeros_np, ones_ch)

  h, a2 = pl.pallas_call(
      _tc_c,
      out_shape=(jax.ShapeDtypeStruct((NP, H), jnp.float32),
                 jax.ShapeDtypeStruct((NP, H), jnp.float32)),
  )(xw10, dis, g1p)

  g2p = _sc_gather_scatter(a2, row2d, col2d, zeros_np, ones_ch)

  out = pl.pallas_call(
      _tc_d,
      out_shape=jax.ShapeDtypeStruct((NP, F), jnp.float32),
  )(h, dis, g2p, W2_0, W2_1, b2r)

  return out[:N]


# speedup vs baseline: 30.0692x; 1.0469x over previous
import functools

import jax
import jax.numpy as jnp
from jax import lax
from jax.experimental import pallas as pl
from jax.experimental.pallas import tpu as pltpu
from jax.experimental.pallas import tpu_sc as plsc

N = 10000
F = 128
H = 16
E = 320000

NC, NS = 2, 16
NW = NC * NS
NP = 10240
SLAB = NP // NS

CH = 512
GRP = 4
CPT = 20
EPT = CPT * CH
EPAD = NW * EPT
ROWS2D = EPAD // CH


def _sc_scatter_body(with_gather, table_hbm, src2d, dst2d, zeros_hbm,
                     ones_hbm, out_hbm, ridx_v, cidx_v, rows_v, gsem, acc):
  cid = lax.axis_index("c")
  sid = lax.axis_index("s")
  wid = sid * NC + cid

  def body():
    pltpu.sync_copy(zeros_hbm.at[pl.ds(sid * SLAB, SLAB)],
                    acc.at[pl.ds(sid * SLAB, SLAB)])
    if with_gather:
      pltpu.sync_copy(src2d.at[pl.ds(wid * CPT, CPT)], ridx_v)
    else:
      for b in range(GRP):
        pltpu.sync_copy(ones_hbm, rows_v.at[pl.ds(b * CH, CH)])
    pltpu.sync_copy(dst2d.at[pl.ds(wid * CPT, CPT)], cidx_v)
    plsc.subcore_barrier()

    def group(g, _):
      if with_gather:
        cps = []
        for b in range(GRP):
          cps.append(pltpu.async_copy(
              table_hbm.at[ridx_v.at[g * GRP + b]],
              rows_v.at[pl.ds(b * CH, CH)], gsem.at[b]))
        for b in range(GRP):
          cps[b].wait()
          pltpu.sync_copy(rows_v.at[pl.ds(b * CH, CH)],
                          acc.at[cidx_v.at[g * GRP + b]], add=True)
      else:
        for b in range(GRP):
          pltpu.sync_copy(rows_v.at[pl.ds(b * CH, CH)],
                          acc.at[cidx_v.at[g * GRP + b]], add=True)
      return 0

    lax.fori_loop(0, CPT // GRP, group, 0)

    plsc.subcore_barrier()
    pltpu.sync_copy(acc.at[pl.ds(sid * SLAB, SLAB)],
                    out_hbm.at[cid, pl.ds(sid * SLAB, SLAB)])

  body()


def _make_sc_pass(with_gather):
  mesh = plsc.VectorSubcoreMesh(core_axis_name="c", subcore_axis_name="s",
                                num_cores=NC, num_subcores=NS)
  return pl.kernel(
      functools.partial(_sc_scatter_body, with_gather),
      out_type=jax.ShapeDtypeStruct((NC, NP, H), jnp.float32),
      mesh=mesh,
      scratch_types=[
          pltpu.VMEM((CPT, CH), jnp.int32),
          pltpu.VMEM((CPT, CH), jnp.int32),
          pltpu.VMEM((GRP * CH, H), jnp.float32),
          pltpu.SemaphoreType.DMA((GRP,)),
          pltpu.VMEM_SHARED((NP, H), jnp.float32),
      ],
      compiler_params=pltpu.CompilerParams(use_tc_tiling_on_sc=False),
  )


_sc_gather_scatter = _make_sc_pass(True)
_sc_degree = _make_sc_pass(False)


def _tc_a(x_ref, w0_ref, w1_ref, b1_ref, xw10_ref, xw11_ref):
  x = x_ref[...]
  xw10_ref[...] = jnp.dot(x, w0_ref[...],
                          preferred_element_type=jnp.float32) + b1_ref[...]
  xw11_ref[...] = jnp.dot(x, w1_ref[...], preferred_element_type=jnp.float32)


def _tc_b(degp_ref, xw11_ref, dis_ref, a1_ref):
  deg = degp_ref[0] + degp_ref[1]
  dis = jnp.where(deg > 0.0, lax.rsqrt(deg), 0.0)
  dis_ref[...] = dis
  a1_ref[...] = dis * xw11_ref[...]


def _tc_c(xw10_ref, dis_ref, g1p_ref, h_ref, a2_ref):
  g1 = g1p_ref[0] + g1p_ref[1]
  h = jnp.maximum(xw10_ref[...] - dis_ref[...] * g1, 0.0)
  h_ref[...] = h
  a2_ref[...] = dis_ref[...] * h


def _tc_d(h_ref, dis_ref, g2p_ref, w20_ref, w21_ref, b2_ref, out_ref):
  t = -(dis_ref[...] * (g2p_ref[0] + g2p_ref[1]))
  o = (jnp.dot(h_ref[...], w20_ref[...], preferred_element_type=jnp.float32)
       + jnp.dot(t, w21_ref[...], preferred_element_type=jnp.float32)
       + b2_ref[...])
  m = jnp.max(o, axis=1, keepdims=True)
  s = jnp.sum(jnp.exp(o - m), axis=1, keepdims=True)
  out_ref[...] = (o - m) - jnp.log(s)


def kernel(x, edge_index, W1_0, W1_1, b1, W2_0, W2_1, b2):
  x_pad = jnp.concatenate(
      [x, jnp.zeros((NP - N, F), jnp.float32)], axis=0)
  epad = jnp.full((2, EPAD - E), N, jnp.int32)
  ei = jnp.concatenate([edge_index, epad], axis=1)
  row2d = ei[0].reshape(ROWS2D, CH)
  col2d = ei[1].reshape(ROWS2D, CH)
  zeros_np = jnp.zeros((NP, H), jnp.float32)
  ones_ch = jnp.ones((CH, H), jnp.float32)
  b1r = b1.reshape(1, H)
  b2r = b2.reshape(1, F)

  xw10, xw11 = pl.pallas_call(
      _tc_a,
      out_shape=(jax.ShapeDtypeStruct((NP, H), jnp.float32),
                 jax.ShapeDtypeStruct((NP, H), jnp.float32)),
  )(x_pad, W1_0, W1_1, b1r)

  degp = _sc_degree(zeros_np, row2d, row2d, zeros_np, ones_ch)

  dis, a1 = pl.pallas_call(
      _tc_b,
      out_shape=(jax.ShapeDtypeStruct((NP, H), jnp.float32),
                 jax.ShapeDtypeStruct((NP, H), jnp.float32)),
  )(degp, xw11)

  g1p = _sc_gather_scatter(a1, row2d, col2d, zeros_np, ones_ch)

  h, a2 = pl.pallas_call(
      _tc_c,
      out_shape=(jax.ShapeDtypeStruct((NP, H), jnp.float32),
                 jax.ShapeDtypeStruct((NP, H), jnp.float32)),
  )(xw10, dis, g1p)

  g2p = _sc_gather_scatter(a2, row2d, col2d, zeros_np, ones_ch)

  out = pl.pallas_call(
      _tc_d,
      out_shape=jax.ShapeDtypeStruct((NP, F), jnp.float32),
  )(h, dis, g2p, W2_0, W2_1, b2r)

  return out[:N]


# speedup vs baseline: 30.0985x; 1.0010x over previous
import functools

import jax
import jax.numpy as jnp
from jax import lax
from jax.experimental import pallas as pl
from jax.experimental.pallas import tpu as pltpu
from jax.experimental.pallas import tpu_sc as plsc

N = 10000
F = 128
H = 16
E = 320000

NC, NS = 2, 16
NW = NC * NS
NP = 10240
SLAB = NP // NS

CH = 512
GRP = 4
CPT = 20
EPT = CPT * CH
EPAD = NW * EPT
ROWS2D = EPAD // CH


def _sc_scatter_body(with_gather, table_hbm, src2d, dst2d, zeros_hbm,
                     ones_hbm, out_hbm, ridx_v, cidx_v, rows_v, gsem, ssem,
                     acc):
  cid = lax.axis_index("c")
  sid = lax.axis_index("s")
  wid = sid * NC + cid

  def body():
    pltpu.sync_copy(zeros_hbm.at[pl.ds(sid * SLAB, SLAB)],
                    acc.at[pl.ds(sid * SLAB, SLAB)])
    if with_gather:
      pltpu.sync_copy(src2d.at[pl.ds(wid * CPT, CPT)], ridx_v)
    else:
      for b in range(GRP):
        pltpu.sync_copy(ones_hbm, rows_v.at[pl.ds(b * CH, CH)])
    pltpu.sync_copy(dst2d.at[pl.ds(wid * CPT, CPT)], cidx_v)
    plsc.subcore_barrier()

    def group(g, _):
      scs = []
      if with_gather:
        cps = []
        for b in range(GRP):
          cps.append(pltpu.async_copy(
              table_hbm.at[ridx_v.at[g * GRP + b]],
              rows_v.at[pl.ds(b * CH, CH)], gsem.at[b]))
        for b in range(GRP):
          cps[b].wait()
          scs.append(pltpu.async_copy(
              rows_v.at[pl.ds(b * CH, CH)],
              acc.at[cidx_v.at[g * GRP + b]], ssem.at[b], add=True))
      else:
        for b in range(GRP):
          scs.append(pltpu.async_copy(
              rows_v.at[pl.ds(b * CH, CH)],
              acc.at[cidx_v.at[g * GRP + b]], ssem.at[b], add=True))
      for b in range(GRP):
        scs[b].wait()
      return 0

    lax.fori_loop(0, CPT // GRP, group, 0)

    plsc.subcore_barrier()
    pltpu.sync_copy(acc.at[pl.ds(sid * SLAB, SLAB)],
                    out_hbm.at[cid, pl.ds(sid * SLAB, SLAB)])

  body()


def _make_sc_pass(with_gather):
  mesh = plsc.VectorSubcoreMesh(core_axis_name="c", subcore_axis_name="s",
                                num_cores=NC, num_subcores=NS)
  return pl.kernel(
      functools.partial(_sc_scatter_body, with_gather),
      out_type=jax.ShapeDtypeStruct((NC, NP, H), jnp.float32),
      mesh=mesh,
      scratch_types=[
          pltpu.VMEM((CPT, CH), jnp.int32),
          pltpu.VMEM((CPT, CH), jnp.int32),
          pltpu.VMEM((GRP * CH, H), jnp.float32),
          pltpu.SemaphoreType.DMA((GRP,)),
          pltpu.SemaphoreType.DMA((GRP,)),
          pltpu.VMEM_SHARED((NP, H), jnp.float32),
      ],
      compiler_params=pltpu.CompilerParams(use_tc_tiling_on_sc=False),
  )


_sc_gather_scatter = _make_sc_pass(True)
_sc_degree = _make_sc_pass(False)


def _tc_a(x_ref, w0_ref, w1_ref, b1_ref, xw10_ref, xw11_ref):
  x = x_ref[...]
  xw10_ref[...] = jnp.dot(x, w0_ref[...],
                          preferred_element_type=jnp.float32) + b1_ref[...]
  xw11_ref[...] = jnp.dot(x, w1_ref[...], preferred_element_type=jnp.float32)


def _tc_b(degp_ref, xw11_ref, dis_ref, a1_ref):
  deg = degp_ref[0] + degp_ref[1]
  dis = jnp.where(deg > 0.0, lax.rsqrt(deg), 0.0)
  dis_ref[...] = dis
  a1_ref[...] = dis * xw11_ref[...]


def _tc_c(xw10_ref, dis_ref, g1p_ref, h_ref, a2_ref):
  g1 = g1p_ref[0] + g1p_ref[1]
  h = jnp.maximum(xw10_ref[...] - dis_ref[...] * g1, 0.0)
  h_ref[...] = h
  a2_ref[...] = dis_ref[...] * h


def _tc_d(h_ref, dis_ref, g2p_ref, w20_ref, w21_ref, b2_ref, out_ref):
  t = -(dis_ref[...] * (g2p_ref[0] + g2p_ref[1]))
  o = (jnp.dot(h_ref[...], w20_ref[...], preferred_element_type=jnp.float32)
       + jnp.dot(t, w21_ref[...], preferred_element_type=jnp.float32)
       + b2_ref[...])
  m = jnp.max(o, axis=1, keepdims=True)
  s = jnp.sum(jnp.exp(o - m), axis=1, keepdims=True)
  out_ref[...] = (o - m) - jnp.log(s)


def kernel(x, edge_index, W1_0, W1_1, b1, W2_0, W2_1, b2):
  x_pad = jnp.concatenate(
      [x, jnp.zeros((NP - N, F), jnp.float32)], axis=0)
  epad = jnp.full((2, EPAD - E), N, jnp.int32)
  ei = jnp.concatenate([edge_index, epad], axis=1)
  row2d = ei[0].reshape(ROWS2D, CH)
  col2d = ei[1].reshape(ROWS2D, CH)
  zeros_np = jnp.zeros((NP, H), jnp.float32)
  ones_ch = jnp.ones((CH, H), jnp.float32)
  b1r = b1.reshape(1, H)
  b2r = b2.reshape(1, F)

  xw10, xw11 = pl.pallas_call(
      _tc_a,
      out_shape=(jax.ShapeDtypeStruct((NP, H), jnp.float32),
                 jax.ShapeDtypeStruct((NP, H), jnp.float32)),
  )(x_pad, W1_0, W1_1, b1r)

  degp = _sc_degree(zeros_np, row2d, row2d, zeros_np, ones_ch)

  dis, a1 = pl.pallas_call(
      _tc_b,
      out_shape=(jax.ShapeDtypeStruct((NP, H), jnp.float32),
                 jax.ShapeDtypeStruct((NP, H), jnp.float32)),
  )(degp, xw11)

  g1p = _sc_gather_scatter(a1, row2d, col2d, zeros_np, ones_ch)

  h, a2 = pl.pallas_call(
      _tc_c,
      out_shape=(jax.ShapeDtypeStruct((NP, H), jnp.float32),
                 jax.ShapeDtypeStruct((NP, H), jnp.float32)),
  )(xw10, dis, g1p)

  g2p = _sc_gather_scatter(a2, row2d, col2d, zeros_np, ones_ch)

  out = pl.pallas_call(
      _tc_d,
      out_shape=jax.ShapeDtypeStruct((NP, F), jnp.float32),
  )(h, dis, g2p, W2_0, W2_1, b2r)

  return out[:N]


# speedup vs baseline: 30.7767x; 1.0225x over previous
import functools

import jax
import jax.numpy as jnp
from jax import lax
from jax.experimental import pallas as pl
from jax.experimental.pallas import tpu as pltpu
from jax.experimental.pallas import tpu_sc as plsc

N = 10000
F = 128
H = 16
E = 320000

NC, NS = 2, 16
NW = NC * NS
NP = 10240
SLAB = NP // NS

CH = 512
GRP = 4
CPT = 20
EPT = CPT * CH
EPAD = NW * EPT
ROWS2D = EPAD // CH


def _sc_scatter_body(with_gather, table_hbm, src2d, dst2d, zeros_hbm,
                     ones_hbm, out_hbm, ridx_v, cidx_v, rows_v, gsem, ssem,
                     acc):
  cid = lax.axis_index("c")
  sid = lax.axis_index("s")
  wid = sid * NC + cid

  def body():
    pltpu.sync_copy(zeros_hbm.at[pl.ds(sid * SLAB, SLAB)],
                    acc.at[pl.ds(sid * SLAB, SLAB)])
    if with_gather:
      pltpu.sync_copy(src2d.at[pl.ds(wid * CPT, CPT)], ridx_v)
    else:
      for b in range(GRP):
        pltpu.sync_copy(ones_hbm, rows_v.at[pl.ds(b * CH, CH)])
    pltpu.sync_copy(dst2d.at[pl.ds(wid * CPT, CPT)], cidx_v)
    plsc.subcore_barrier()

    def group(g, _):
      scs = []
      if with_gather:
        cps = []
        for b in range(GRP):
          cps.append(pltpu.async_copy(
              table_hbm.at[ridx_v.at[g * GRP + b]],
              rows_v.at[pl.ds(b * CH, CH)], gsem.at[b]))
        for b in range(GRP):
          cps[b].wait()
          scs.append(pltpu.async_copy(
              rows_v.at[pl.ds(b * CH, CH)],
              acc.at[cidx_v.at[g * GRP + b]], ssem.at[b], add=True))
      else:
        for b in range(GRP):
          scs.append(pltpu.async_copy(
              rows_v.at[pl.ds(b * CH, CH)],
              acc.at[cidx_v.at[g * GRP + b]], ssem.at[b], add=True))
      for b in range(GRP):
        scs[b].wait()
      return 0

    lax.fori_loop(0, CPT // GRP, group, 0)

    plsc.subcore_barrier()
    pltpu.sync_copy(acc.at[pl.ds(sid * SLAB, SLAB)],
                    out_hbm.at[cid, pl.ds(sid * SLAB, SLAB)])

  body()


def _make_sc_pass(with_gather):
  mesh = plsc.VectorSubcoreMesh(core_axis_name="c", subcore_axis_name="s",
                                num_cores=NC, num_subcores=NS)
  return pl.kernel(
      functools.partial(_sc_scatter_body, with_gather),
      out_type=jax.ShapeDtypeStruct((NC, NP, H), jnp.float32),
      mesh=mesh,
      scratch_types=[
          pltpu.VMEM((CPT, CH), jnp.int32),
          pltpu.VMEM((CPT, CH), jnp.int32),
          pltpu.VMEM((GRP * CH, H), jnp.float32),
          pltpu.SemaphoreType.DMA((GRP,)),
          pltpu.SemaphoreType.DMA((GRP,)),
          pltpu.VMEM_SHARED((NP, H), jnp.float32),
      ],
      compiler_params=pltpu.CompilerParams(use_tc_tiling_on_sc=False),
  )


_sc_gather_scatter = _make_sc_pass(True)


def _sc_degree_body(dst2d, zeros1_hbm, ones1_hbm, out_hbm, cidx_v, ones_v,
                    ssem, acc):
  cid = lax.axis_index("c")
  sid = lax.axis_index("s")
  wid = sid * NC + cid

  pltpu.sync_copy(zeros1_hbm.at[pl.ds(sid * SLAB, SLAB)],
                  acc.at[pl.ds(sid * SLAB, SLAB)])
  pltpu.sync_copy(ones1_hbm, ones_v)
  pltpu.sync_copy(dst2d.at[pl.ds(wid * CPT, CPT)], cidx_v)
  plsc.subcore_barrier()

  def group(g, _):
    scs = []
    for b in range(GRP):
      scs.append(pltpu.async_copy(
          ones_v, acc.at[cidx_v.at[g * GRP + b]], ssem.at[b], add=True))
    for b in range(GRP):
      scs[b].wait()
    return 0

  lax.fori_loop(0, CPT // GRP, group, 0)

  plsc.subcore_barrier()
  pltpu.sync_copy(acc.at[pl.ds(sid * SLAB, SLAB)],
                  out_hbm.at[cid, pl.ds(sid * SLAB, SLAB)])


_sc_degree = pl.kernel(
    _sc_degree_body,
    out_type=jax.ShapeDtypeStruct((NC, NP), jnp.float32),
    mesh=plsc.VectorSubcoreMesh(core_axis_name="c", subcore_axis_name="s",
                                num_cores=NC, num_subcores=NS),
    scratch_types=[
        pltpu.VMEM((CPT, CH), jnp.int32),
        pltpu.VMEM((CH,), jnp.float32),
        pltpu.SemaphoreType.DMA((GRP,)),
        pltpu.VMEM_SHARED((NP,), jnp.float32),
    ],
    compiler_params=pltpu.CompilerParams(use_tc_tiling_on_sc=False),
)


def _tc_a(x_ref, w0_ref, w1_ref, b1_ref, xw10_ref, xw11_ref):
  x = x_ref[...]
  xw10_ref[...] = jnp.dot(x, w0_ref[...],
                          preferred_element_type=jnp.float32) + b1_ref[...]
  xw11_ref[...] = jnp.dot(x, w1_ref[...], preferred_element_type=jnp.float32)


def _tc_b(degt_ref, xw11_ref, dis_ref, a1_ref):
  deg = degt_ref[:, 0:1] + degt_ref[:, 1:2]
  dis = jnp.where(deg > 0.0, lax.rsqrt(deg), 0.0)
  dis16 = jnp.broadcast_to(dis, (NP, H))
  dis_ref[...] = dis16
  a1_ref[...] = dis16 * xw11_ref[...]


def _tc_c(xw10_ref, dis_ref, g1p_ref, h_ref, a2_ref):
  g1 = g1p_ref[0] + g1p_ref[1]
  h = jnp.maximum(xw10_ref[...] - dis_ref[...] * g1, 0.0)
  h_ref[...] = h
  a2_ref[...] = dis_ref[...] * h


def _tc_d(h_ref, dis_ref, g2p_ref, w20_ref, w21_ref, b2_ref, out_ref):
  t = -(dis_ref[...] * (g2p_ref[0] + g2p_ref[1]))
  o = (jnp.dot(h_ref[...], w20_ref[...], preferred_element_type=jnp.float32)
       + jnp.dot(t, w21_ref[...], preferred_element_type=jnp.float32)
       + b2_ref[...])
  m = jnp.max(o, axis=1, keepdims=True)
  s = jnp.sum(jnp.exp(o - m), axis=1, keepdims=True)
  out_ref[...] = (o - m) - jnp.log(s)


def kernel(x, edge_index, W1_0, W1_1, b1, W2_0, W2_1, b2):
  x_pad = jnp.concatenate(
      [x, jnp.zeros((NP - N, F), jnp.float32)], axis=0)
  epad = jnp.full((2, EPAD - E), N, jnp.int32)
  ei = jnp.concatenate([edge_index, epad], axis=1)
  row2d = ei[0].reshape(ROWS2D, CH)
  col2d = ei[1].reshape(ROWS2D, CH)
  zeros_np = jnp.zeros((NP, H), jnp.float32)
  ones_ch = jnp.ones((CH, H), jnp.float32)
  b1r = b1.reshape(1, H)
  b2r = b2.reshape(1, F)

  xw10, xw11 = pl.pallas_call(
      _tc_a,
      out_shape=(jax.ShapeDtypeStruct((NP, H), jnp.float32),
                 jax.ShapeDtypeStruct((NP, H), jnp.float32)),
  )(x_pad, W1_0, W1_1, b1r)

  degp = _sc_degree(row2d, jnp.zeros((NP,), jnp.float32),
                    jnp.ones((CH,), jnp.float32))

  dis, a1 = pl.pallas_call(
      _tc_b,
      out_shape=(jax.ShapeDtypeStruct((NP, H), jnp.float32),
                 jax.ShapeDtypeStruct((NP, H), jnp.float32)),
  )(degp.T, xw11)

  g1p = _sc_gather_scatter(a1, row2d, col2d, zeros_np, ones_ch)

  h, a2 = pl.pallas_call(
      _tc_c,
      out_shape=(jax.ShapeDtypeStruct((NP, H), jnp.float32),
                 jax.ShapeDtypeStruct((NP, H), jnp.float32)),
  )(xw10, dis, g1p)

  g2p = _sc_gather_scatter(a2, row2d, col2d, zeros_np, ones_ch)

  out = pl.pallas_call(
      _tc_d,
      out_shape=jax.ShapeDtypeStruct((NP, F), jnp.float32),
  )(h, dis, g2p, W2_0, W2_1, b2r)

  return out[:N]


# speedup vs baseline: 43.6261x; 1.4175x over previous
import jax
import jax.numpy as jnp
from jax import lax
from jax.experimental import pallas as pl
from jax.experimental.pallas import tpu as pltpu
from jax.experimental.pallas import tpu_sc as plsc

N = 10000
F = 128
H = 16
E = 320000

NC, NS = 2, 16
NP = 10240
SLAB = NP // NS

CH = 512
GRP = 4
CPT = 40
NG = CPT // GRP
EPAD = NS * CPT * CH
ROWS2D = EPAD // CH


def _rsqrt16(d):
  i = plsc.bitcast(d, jnp.int32)
  y = plsc.bitcast(jnp.int32(0x5F3759DF) - (i >> 1), jnp.float32)
  for _ in range(3):
    y = y * (1.5 - 0.5 * d * y * y)
  return jnp.where(d > 0.5, y, 0.0)


def _splat(ref, i):
  return plsc.load_gather(ref, [jnp.full((16,), i, jnp.int32)])


def _sc_fused_body(row2d, col2d, xw10, xw11, zeros16, zeros1, ones1,
                   h_hbm, t_hbm,
                   ridx, cidx, rows, sb_x, sb_g, sb_deg, sb_dis, ones_v,
                   gsem, ssem, deg_acc, a_sp, g_acc):
  cid = lax.axis_index("c")
  sid = lax.axis_index("s")
  nsl = pl.ds(sid * SLAB, SLAB)
  ebase = sid * CPT

  pltpu.sync_copy(zeros1.at[nsl], deg_acc.at[nsl])
  pltpu.sync_copy(zeros16.at[nsl], g_acc.at[nsl])
  pltpu.sync_copy(ones1, ones_v)
  pltpu.sync_copy(row2d.at[pl.ds(ebase, CPT)], ridx)
  pltpu.sync_copy(col2d.at[pl.ds(ebase, CPT)], cidx)
  plsc.subcore_barrier()

  def deg_group(g, _):
    scs = [pltpu.async_copy(ones_v, deg_acc.at[ridx.at[g * GRP + b]],
                            ssem.at[b], add=True) for b in range(GRP)]
    for b in range(GRP):
      scs[b].wait()
    return 0
  lax.fori_loop(0, NG, deg_group, 0)
  plsc.subcore_barrier()

  pltpu.sync_copy(deg_acc.at[nsl], sb_deg)
  pltpu.sync_copy(xw11.at[nsl], sb_x)

  def dis_vec(i, _):
    sb_dis[pl.ds(i * 16, 16)] = _rsqrt16(sb_deg[pl.ds(i * 16, 16)])
    return 0
  lax.fori_loop(0, SLAB // 16, dis_vec, 0)

  def a1_row(i, _):
    sb_x[i] = _splat(sb_dis, i) * sb_x[i]
    return 0
  lax.fori_loop(0, SLAB, a1_row, 0)
  pltpu.sync_copy(sb_x, a_sp.at[nsl])
  plsc.subcore_barrier()

  def gs_group(g, _):
    gcs = [pltpu.async_copy(a_sp.at[ridx.at[g * GRP + b]],
                            rows.at[pl.ds(b * CH, CH)], gsem.at[b])
           for b in range(GRP)]
    scs = []
    for b in range(GRP):
      gcs[b].wait()
      scs.append(pltpu.async_copy(rows.at[pl.ds(b * CH, CH)],
                                  g_acc.at[cidx.at[g * GRP + b]],
                                  ssem.at[b], add=True))
    for b in range(GRP):
      scs[b].wait()
    return 0

  lax.fori_loop(0, NG, gs_group, 0)
  plsc.subcore_barrier()

  pltpu.sync_copy(g_acc.at[nsl], sb_g)
  pltpu.sync_copy(xw10.at[nsl], sb_x)

  def h_row(i, _):
    disv = _splat(sb_dis, i)
    hv = jnp.maximum(sb_x[i] - disv * sb_g[i], 0.0)
    sb_x[i] = hv
    sb_g[i] = disv * hv
    return 0
  lax.fori_loop(0, SLAB, h_row, 0)

  pltpu.sync_copy(zeros16.at[nsl], g_acc.at[nsl])
  pltpu.sync_copy(sb_g, a_sp.at[nsl])
  @pl.when(sid // (NS // NC) == cid)
  def _():
    pltpu.sync_copy(sb_x, h_hbm.at[nsl])
  plsc.subcore_barrier()

  lax.fori_loop(0, NG, gs_group, 0)
  plsc.subcore_barrier()

  pltpu.sync_copy(g_acc.at[nsl], sb_g)

  def t_row(i, _):
    sb_g[i] = -(_splat(sb_dis, i) * sb_g[i])
    return 0
  lax.fori_loop(0, SLAB, t_row, 0)

  @pl.when(sid // (NS // NC) == cid)
  def _():
    pltpu.sync_copy(sb_g, t_hbm.at[nsl])


_sc_fused = pl.kernel(
    _sc_fused_body,
    out_type=(jax.ShapeDtypeStruct((NP, H), jnp.float32),
              jax.ShapeDtypeStruct((NP, H), jnp.float32)),
    mesh=plsc.VectorSubcoreMesh(core_axis_name="c", subcore_axis_name="s",
                                num_cores=NC, num_subcores=NS),
    scratch_types=[
        pltpu.VMEM((CPT, CH), jnp.int32),
        pltpu.VMEM((CPT, CH), jnp.int32),
        pltpu.VMEM((GRP * CH, H), jnp.float32),
        pltpu.VMEM((SLAB, H), jnp.float32),
        pltpu.VMEM((SLAB, H), jnp.float32),
        pltpu.VMEM((SLAB,), jnp.float32),
        pltpu.VMEM((SLAB,), jnp.float32),
        pltpu.VMEM((CH,), jnp.float32),
        pltpu.SemaphoreType.DMA((GRP,)),
        pltpu.SemaphoreType.DMA((GRP,)),
        pltpu.VMEM_SHARED((NP,), jnp.float32),
        pltpu.VMEM_SHARED((NP, H), jnp.float32),
        pltpu.VMEM_SHARED((NP, H), jnp.float32),
    ],
    compiler_params=pltpu.CompilerParams(use_tc_tiling_on_sc=False,
                                         needs_layout_passes=False),
)


def _tc_a(x_ref, w0_ref, w1_ref, b1_ref, xw10_ref, xw11_ref):
  x = x_ref[...]
  xw10_ref[...] = jnp.dot(x, w0_ref[...],
                          preferred_element_type=jnp.float32) + b1_ref[...]
  xw11_ref[...] = jnp.dot(x, w1_ref[...], preferred_element_type=jnp.float32)


def _tc_d(h_ref, t_ref, w20_ref, w21_ref, b2_ref, out_ref):
  o = (jnp.dot(h_ref[...], w20_ref[...], preferred_element_type=jnp.float32)
       + jnp.dot(t_ref[...], w21_ref[...], preferred_element_type=jnp.float32)
       + b2_ref[...])
  m = jnp.max(o, axis=1, keepdims=True)
  s = jnp.sum(jnp.exp(o - m), axis=1, keepdims=True)
  out_ref[...] = (o - m) - jnp.log(s)


def kernel(x, edge_index, W1_0, W1_1, b1, W2_0, W2_1, b2):
  x_pad = jnp.concatenate(
      [x, jnp.zeros((NP - N, F), jnp.float32)], axis=0)
  epad = jnp.full((2, EPAD - E), N, jnp.int32)
  ei = jnp.concatenate([edge_index, epad], axis=1)
  row2d = ei[0].reshape(ROWS2D, CH)
  col2d = ei[1].reshape(ROWS2D, CH)

  xw10, xw11 = pl.pallas_call(
      _tc_a,
      out_shape=(jax.ShapeDtypeStruct((NP, H), jnp.float32),
                 jax.ShapeDtypeStruct((NP, H), jnp.float32)),
  )(x_pad, W1_0, W1_1, b1.reshape(1, H))

  h, t = _sc_fused(row2d, col2d, xw10, xw11,
                   jnp.zeros((NP, H), jnp.float32),
                   jnp.zeros((NP,), jnp.float32),
                   jnp.ones((CH,), jnp.float32))

  out = pl.pallas_call(
      _tc_d,
      out_shape=jax.ShapeDtypeStruct((NP, F), jnp.float32),
  )(h, t, W2_0, W2_1, b2.reshape(1, F))

  return out[:N]


# speedup vs baseline: 53.1631x; 1.2186x over previous
import jax
import jax.numpy as jnp
from jax import lax
from jax.experimental import pallas as pl
from jax.experimental.pallas import tpu as pltpu
from jax.experimental.pallas import tpu_sc as plsc

N = 10000
F = 128
H = 16
E = 320000

NC, NS = 2, 16
NP = 10240
SLAB = NP // NS

CH = 625
GRP = 4
CPT = 32
NG = CPT // GRP
ROWS2D = E // CH


def _rsqrt16(d):
  i = plsc.bitcast(d, jnp.int32)
  y = plsc.bitcast(jnp.int32(0x5F3759DF) - (i >> 1), jnp.float32)
  for _ in range(3):
    y = y * (1.5 - 0.5 * d * y * y)
  return jnp.where(d > 0.5, y, 0.0)


def _splat(ref, i):
  return plsc.load_gather(ref, [jnp.full((16,), i, jnp.int32)])


def _sc_fused_body(ei3, xw10, xw11, zeros16, zeros1, ones1,
                   h_hbm, t_hbm,
                   ridx, cidx, rows, sb_x, sb_g, sb_deg, sb_dis, ones_v,
                   gsem, ssem, deg_acc, a_sp, g_acc):
  cid = lax.axis_index("c")
  sid = lax.axis_index("s")
  nsl = pl.ds(sid * SLAB, SLAB)
  ebase = sid * CPT

  pltpu.sync_copy(zeros1.at[nsl], deg_acc.at[nsl])
  pltpu.sync_copy(zeros16.at[nsl], g_acc.at[nsl])
  pltpu.sync_copy(ones1, ones_v)
  pltpu.sync_copy(ei3.at[0, pl.ds(ebase, CPT)], ridx)
  pltpu.sync_copy(ei3.at[1, pl.ds(ebase, CPT)], cidx)
  plsc.subcore_barrier()

  def deg_group(g, _):
    scs = [pltpu.async_copy(ones_v, deg_acc.at[ridx.at[g * GRP + b]],
                            ssem.at[b], add=True) for b in range(GRP)]
    for b in range(GRP):
      scs[b].wait()
    return 0
  lax.fori_loop(0, NG, deg_group, 0)
  plsc.subcore_barrier()

  pltpu.sync_copy(deg_acc.at[nsl], sb_deg)
  pltpu.sync_copy(xw11.at[nsl], sb_x)

  def dis_vec(i, _):
    sb_dis[pl.ds(i * 16, 16)] = _rsqrt16(sb_deg[pl.ds(i * 16, 16)])
    return 0
  lax.fori_loop(0, SLAB // 16, dis_vec, 0)

  def a1_row(i, _):
    sb_x[i] = _splat(sb_dis, i) * sb_x[i]
    return 0
  lax.fori_loop(0, SLAB, a1_row, 0)
  pltpu.sync_copy(sb_x, a_sp.at[nsl])
  plsc.subcore_barrier()

  def gs_group(g, _):
    gcs = [pltpu.async_copy(a_sp.at[ridx.at[g * GRP + b]],
                            rows.at[pl.ds(b * CH, CH)], gsem.at[b])
           for b in range(GRP)]
    scs = []
    for b in range(GRP):
      gcs[b].wait()
      scs.append(pltpu.async_copy(rows.at[pl.ds(b * CH, CH)],
                                  g_acc.at[cidx.at[g * GRP + b]],
                                  ssem.at[b], add=True))
    for b in range(GRP):
      scs[b].wait()
    return 0

  lax.fori_loop(0, NG, gs_group, 0)
  plsc.subcore_barrier()

  pltpu.sync_copy(g_acc.at[nsl], sb_g)
  pltpu.sync_copy(xw10.at[nsl], sb_x)

  def h_row(i, _):
    disv = _splat(sb_dis, i)
    hv = jnp.maximum(sb_x[i] - disv * sb_g[i], 0.0)
    sb_x[i] = hv
    sb_g[i] = disv * hv
    return 0
  lax.fori_loop(0, SLAB, h_row, 0)

  pltpu.sync_copy(zeros16.at[nsl], g_acc.at[nsl])
  pltpu.sync_copy(sb_g, a_sp.at[nsl])
  @pl.when(sid // (NS // NC) == cid)
  def _():
    pltpu.sync_copy(sb_x, h_hbm.at[nsl])
  plsc.subcore_barrier()

  lax.fori_loop(0, NG, gs_group, 0)
  plsc.subcore_barrier()

  pltpu.sync_copy(g_acc.at[nsl], sb_g)

  def t_row(i, _):
    sb_g[i] = -(_splat(sb_dis, i) * sb_g[i])
    return 0
  lax.fori_loop(0, SLAB, t_row, 0)

  @pl.when(sid // (NS // NC) == cid)
  def _():
    pltpu.sync_copy(sb_g, t_hbm.at[nsl])


_sc_fused = pl.kernel(
    _sc_fused_body,
    out_type=(jax.ShapeDtypeStruct((NP, H), jnp.float32),
              jax.ShapeDtypeStruct((NP, H), jnp.float32)),
    mesh=plsc.VectorSubcoreMesh(core_axis_name="c", subcore_axis_name="s",
                                num_cores=NC, num_subcores=NS),
    scratch_types=[
        pltpu.VMEM((CPT, CH), jnp.int32),
        pltpu.VMEM((CPT, CH), jnp.int32),
        pltpu.VMEM((GRP * CH, H), jnp.float32),
        pltpu.VMEM((SLAB, H), jnp.float32),
        pltpu.VMEM((SLAB, H), jnp.float32),
        pltpu.VMEM((SLAB,), jnp.float32),
        pltpu.VMEM((SLAB,), jnp.float32),
        pltpu.VMEM((CH,), jnp.float32),
        pltpu.SemaphoreType.DMA((GRP,)),
        pltpu.SemaphoreType.DMA((GRP,)),
        pltpu.VMEM_SHARED((NP,), jnp.float32),
        pltpu.VMEM_SHARED((NP, H), jnp.float32),
        pltpu.VMEM_SHARED((NP, H), jnp.float32),
    ],
    compiler_params=pltpu.CompilerParams(use_tc_tiling_on_sc=False,
                                         needs_layout_passes=False),
)


def _tc_a(x_ref, w0_ref, w1_ref, b1_ref, xw10_ref, xw11_ref):
  x = x_ref[...]
  xw10_ref[:N, :] = jnp.dot(x, w0_ref[...],
                            preferred_element_type=jnp.float32) + b1_ref[...]
  xw11_ref[:N, :] = jnp.dot(x, w1_ref[...], preferred_element_type=jnp.float32)
  zt = jnp.zeros((NP - N, H), jnp.float32)
  xw10_ref[N:, :] = zt
  xw11_ref[N:, :] = zt


def _tc_d(h_ref, t_ref, w20_ref, w21_ref, b2_ref, out_ref):
  h = h_ref[:N, :]
  t = t_ref[:N, :]
  o = (jnp.dot(h, w20_ref[...], preferred_element_type=jnp.float32)
       + jnp.dot(t, w21_ref[...], preferred_element_type=jnp.float32)
       + b2_ref[...])
  m = jnp.max(o, axis=1, keepdims=True)
  s = jnp.sum(jnp.exp(o - m), axis=1, keepdims=True)
  out_ref[...] = (o - m) - jnp.log(s)


def kernel(x, edge_index, W1_0, W1_1, b1, W2_0, W2_1, b2):
  ei3 = edge_index.reshape(2, ROWS2D, CH)

  xw10, xw11 = pl.pallas_call(
      _tc_a,
      out_shape=(jax.ShapeDtypeStruct((NP, H), jnp.float32),
                 jax.ShapeDtypeStruct((NP, H), jnp.float32)),
  )(x, W1_0, W1_1, b1.reshape(1, H))

  h, t = _sc_fused(ei3, xw10, xw11,
                   jnp.zeros((NP, H), jnp.float32),
                   jnp.zeros((NP,), jnp.float32),
                   jnp.ones((CH,), jnp.float32))

  return pl.pallas_call(
      _tc_d,
      out_shape=jax.ShapeDtypeStruct((N, F), jnp.float32),
  )(h, t, W2_0, W2_1, b2.reshape(1, F))
